# baseline (speedup 1.0000x reference)
import numpy as np
import jax
import jax.numpy as jnp
from jax import lax
from jax.experimental import pallas as pl
from jax.experimental.pallas import tpu as pltpu

_OUT_F = 8192
_IN_F = 8192
_BATCH = 256
_TO = 256
_GRID = _OUT_F // _TO
_RB = _TO * 32
_STR = _TO + 1


def _f16_decode(v):
    e = (v >> 10) & 31
    m = v & 1023
    normal = lax.bitcast_convert_type(((e + 112) << 23) | (m << 13), jnp.float32)
    sub = m.astype(jnp.float32) * np.float32(2.0 ** -24)
    r = jnp.where(e == 0, sub, normal)
    return jnp.where((v >> 15) & 1 == 1, -r, r)


def _q4k_kernel(qw_ref, xp_ref, bias_ref, out_ref, wscr0, wscr1):
    qb = qw_ref[...].astype(jnp.int32)
    hdr = qb[:, 0:16]

    s = lax.broadcasted_iota(jnp.int32, (_RB, 8), 1)
    lo4 = s < 4
    ia = jnp.where(lo4, 4 + s, s)
    ib = jnp.where(lo4, 8 + s, 4 + s)
    ic = 12 + (s & 3)
    A = jnp.take_along_axis(hdr, ia, axis=1)
    B = jnp.take_along_axis(hdr, ib, axis=1)
    C = jnp.take_along_axis(hdr, ic, axis=1)
    sc6 = jnp.where(lo4, A & 63, (C & 15) | ((A >> 6) << 4)).astype(jnp.float32)
    mn6 = jnp.where(lo4, B & 63, (C >> 4) | ((B >> 6) << 4)).astype(jnp.float32)
    d = _f16_decode(hdr[:, 0:1] + (hdr[:, 1:2] << 8))
    dmin = _f16_decode(hdr[:, 2:3] + (hdr[:, 3:4] << 8))
    dl8 = d * sc6
    ml8 = dmin * mn6

    li = lax.broadcasted_iota(jnp.int32, (_RB, 2 * 128), 1)
    sidx = 2 * ((li >> 5) & 3) + (li >> 7)
    dl = jnp.take_along_axis(dl8, sidx, axis=1)
    ml = jnp.take_along_axis(ml8, sidx, axis=1)

    q256 = jnp.concatenate(
        [qb[:, 16:144] & 15, (qb[:, 16:144] >> 4) & 15], axis=-1
    ).astype(jnp.float32)
    w = q256 * dl - ml

    for r in range(_TO):
        wscr0[r: r + _STR * 32: _STR, :] = w[32 * r: 32 * r + 32, 0:128]
        wscr1[r: r + _STR * 32: _STR, :] = w[32 * r: 32 * r + 32, 128:256]

    parts = []
    for b in range(32):
        parts.append(wscr0[_STR * b: _STR * b + _TO, :].astype(jnp.bfloat16))
        parts.append(wscr1[_STR * b: _STR * b + _TO, :].astype(jnp.bfloat16))
    wq = jnp.concatenate(parts, axis=-1)

    acc = lax.dot_general(xp_ref[...], wq,
                          ((((1,), (1,)), ((), ()))),
                          preferred_element_type=jnp.float32)
    out_ref[...] = acc + bias_ref[...]


def kernel(x, qweight, bias):
    xp = (x.reshape(_BATCH, 32, 4, 2, 32)
          .transpose(0, 1, 3, 2, 4)
          .reshape(_BATCH, _IN_F)
          .astype(jnp.bfloat16))
    bias_c = bias.reshape(1, _OUT_F)

    out = pl.pallas_call(
        _q4k_kernel,
        grid=(_GRID,),
        in_specs=[
            pl.BlockSpec((_RB, 144), lambda i: (i, 0)),
            pl.BlockSpec((_BATCH, _IN_F), lambda i: (0, 0)),
            pl.BlockSpec((1, _TO), lambda i: (0, i)),
        ],
        out_specs=pl.BlockSpec((_BATCH, _TO), lambda i: (0, i)),
        out_shape=jax.ShapeDtypeStruct((_BATCH, _OUT_F), jnp.float32),
        scratch_shapes=[pltpu.VMEM((_STR * 31 + _TO, 128), jnp.float32),
                        pltpu.VMEM((_STR * 31 + _TO, 128), jnp.float32)],
        compiler_params=pltpu.CompilerParams(
            dimension_semantics=("parallel",)),
    )(qweight, xp, bias_c)

    return out
